# Initial kernel scaffold; baseline (speedup 1.0000x reference)
import jax
import jax.numpy as jnp
from jax.experimental import pallas as pl


def kernel(token_ids, position_ids):
    raise NotImplementedError("write your pallas kernel here")


# speedup vs baseline: 1.9786x; 1.9786x over previous
import functools

import jax
import jax.numpy as jnp
from jax import lax
from jax.experimental import pallas as pl
from jax.experimental.pallas import tpu as pltpu
from jax.experimental.pallas import tpu_sc as plsc

NUM_EMB = 128
MAX_POS = 16
NBINS = NUM_EMB + MAX_POS
LANES = 16
CHARS = 16
NC, NS = 2, 16
NW = NC * NS
W_TOTAL = 16 * 1024
WPT = W_TOTAL // NW

_MESH = plsc.VectorSubcoreMesh(
    core_axis_name="c", subcore_axis_name="s", num_cores=NC, num_subcores=NS
)


@functools.partial(
    pl.kernel,
    out_type=jax.ShapeDtypeStruct((W_TOTAL * NBINS,), jnp.int32),
    mesh=_MESH,
    scratch_types=[
        pltpu.VMEM((WPT * CHARS,), jnp.int32),
        pltpu.VMEM((WPT * CHARS,), jnp.int32),
        pltpu.VMEM((WPT * NBINS,), jnp.int32),
    ],
    compiler_params=pltpu.CompilerParams(needs_layout_passes=False),
)
def _sc_encode(tok_hbm, pos_hbm, out_hbm, tok_v, pos_v, out_v):
    wid = lax.axis_index("s") * NC + lax.axis_index("c")
    base = wid * WPT

    pltpu.sync_copy(tok_hbm.at[pl.ds(base * CHARS, WPT * CHARS)], tok_v)
    pltpu.sync_copy(pos_hbm.at[pl.ds(base * CHARS, WPT * CHARS)], pos_v)

    zeros = jnp.zeros((LANES,), jnp.int32)
    ones = jnp.full((LANES,), 1, jnp.int32)

    def word(w, carry):
        tok = tok_v[pl.ds(w * CHARS, LANES)]
        pos = pos_v[pl.ds(w * CHARS, LANES)]
        row = w * NBINS
        for j in range(NBINS // LANES):
            out_v[pl.ds(row + j * LANES, LANES)] = zeros
        plsc.addupdate_scatter(out_v, [row + tok], ones, mask=tok != 0)
        plsc.addupdate_scatter(
            out_v, [row + NUM_EMB + pos], ones, mask=pos != 0
        )
        return carry

    lax.fori_loop(0, WPT, word, 0)
    pltpu.sync_copy(out_v, out_hbm.at[pl.ds(base * NBINS, WPT * NBINS)])


def kernel(token_ids, position_ids):
    out = _sc_encode(token_ids.reshape(-1), position_ids.reshape(-1))
    return out.reshape(token_ids.shape[0], token_ids.shape[1], NBINS)


# speedup vs baseline: 1.9900x; 1.0057x over previous
import functools

import jax
import jax.numpy as jnp
from jax import lax
from jax.experimental import pallas as pl
from jax.experimental.pallas import tpu as pltpu
from jax.experimental.pallas import tpu_sc as plsc

NUM_EMB = 128
MAX_POS = 16
NBINS = NUM_EMB + MAX_POS
LANES = 16
CHARS = 16
NC, NS = 2, 16
NW = NC * NS
W_TOTAL = 16 * 1024
WPT = W_TOTAL // NW

_MESH = plsc.VectorSubcoreMesh(
    core_axis_name="c", subcore_axis_name="s", num_cores=NC, num_subcores=NS
)


@functools.partial(
    pl.kernel,
    out_type=jax.ShapeDtypeStruct((W_TOTAL * NBINS,), jnp.int32),
    mesh=_MESH,
    scratch_types=[
        pltpu.VMEM((WPT * CHARS,), jnp.int32),
        pltpu.VMEM((WPT * CHARS,), jnp.int32),
        pltpu.VMEM((WPT * NBINS,), jnp.int32),
    ],
    compiler_params=pltpu.CompilerParams(needs_layout_passes=False),
)
def _sc_encode(tok_hbm, pos_hbm, out_hbm, tok_v, pos_v, out_v):
    wid = lax.axis_index("s") * NC + lax.axis_index("c")
    base = wid * WPT

    pltpu.sync_copy(tok_hbm.at[pl.ds(base * CHARS, WPT * CHARS)], tok_v)
    pltpu.sync_copy(pos_hbm.at[pl.ds(base * CHARS, WPT * CHARS)], pos_v)

    zeros = jnp.zeros((LANES,), jnp.int32)
    ones = jnp.full((LANES,), 1, jnp.int32)

    @plsc.parallel_loop(0, WPT, step=1, unroll=8)
    def word(w):
        tok = tok_v[pl.ds(w * CHARS, LANES)]
        pos = pos_v[pl.ds(w * CHARS, LANES)]
        row = w * NBINS
        for j in range(NBINS // LANES):
            out_v[pl.ds(row + j * LANES, LANES)] = zeros
        plsc.addupdate_scatter(out_v, [row + tok], ones, mask=tok != 0)
        plsc.addupdate_scatter(
            out_v, [row + NUM_EMB + pos], ones, mask=pos != 0
        )
    pltpu.sync_copy(out_v, out_hbm.at[pl.ds(base * NBINS, WPT * NBINS)])


def kernel(token_ids, position_ids):
    out = _sc_encode(token_ids.reshape(-1), position_ids.reshape(-1))
    return out.reshape(token_ids.shape[0], token_ids.shape[1], NBINS)


# speedup vs baseline: 5.3996x; 2.7134x over previous
import functools

import jax
import jax.numpy as jnp
from jax import lax
from jax.experimental import pallas as pl
from jax.experimental.pallas import tpu as pltpu
from jax.experimental.pallas import tpu_sc as plsc

NUM_EMB = 128
MAX_POS = 16
NBINS = NUM_EMB + MAX_POS
LANES = 16
CHARS = 16
BATCH = 16
WORDS = 1024
NC, NS = 2, 16
NW = NC * NS
BT = NBINS // 8
WT = WORDS // 128
WTH = WT // 2

_MESH = plsc.VectorSubcoreMesh(
    core_axis_name="c", subcore_axis_name="s", num_cores=NC, num_subcores=NS
)


@functools.partial(
    pl.kernel,
    out_type=jax.ShapeDtypeStruct((BATCH, BT, WT, 8, 128), jnp.int32),
    mesh=_MESH,
    scratch_types=[
        pltpu.VMEM((2, WTH, 8, 128), jnp.int32),
        pltpu.VMEM((2, WTH, 8, 128), jnp.int32),
        pltpu.VMEM((BT, WTH, 8, 128), jnp.int32),
    ],
    compiler_params=pltpu.CompilerParams(needs_layout_passes=False),
)
def _sc_encode(tok_hbm, pos_hbm, out_hbm, tok_v, pos_v, out_v):
    wid = lax.axis_index("s") * NC + lax.axis_index("c")
    b = wid // 2
    wt0 = (wid % 2) * WTH

    pltpu.sync_copy(tok_hbm.at[b, :, pl.ds(wt0, WTH)], tok_v)
    pltpu.sync_copy(pos_hbm.at[b, :, pl.ds(wt0, WTH)], pos_v)

    zeros = jnp.zeros((LANES,), jnp.int32)
    ones = jnp.full((LANES,), 1, jnp.int32)
    iota = jax.lax.iota(jnp.int32, LANES)

    @plsc.parallel_loop(0, WTH * 8, step=1, unroll=1)
    def group(g):
        wt = g // 8
        l0 = (g % 8) * LANES
        lanes = l0 + iota
        wt_vec = jnp.full((LANES,), wt, jnp.int32)
        for bt in range(BT):
            for s in range(8):
                out_v[bt, wt, s, pl.ds(l0, LANES)] = zeros
        for c in range(CHARS):
            tok = tok_v[c // 8, wt, c % 8, pl.ds(l0, LANES)]
            plsc.addupdate_scatter(
                out_v,
                [tok >> 3, wt_vec, tok & 7, lanes],
                ones,
                mask=tok != 0,
            )
            pos = pos_v[c // 8, wt, c % 8, pl.ds(l0, LANES)]
            plsc.addupdate_scatter(
                out_v,
                [(NUM_EMB + pos) >> 3, wt_vec, pos & 7, lanes],
                ones,
                mask=pos != 0,
            )

    pltpu.sync_copy(out_v, out_hbm.at[b, :, pl.ds(wt0, WTH)])


def kernel(token_ids, position_ids):
    def to_tiles(x):
        x = x.transpose(0, 2, 1).reshape(BATCH, 2, 8, WT, 128)
        return x.transpose(0, 1, 3, 2, 4)

    out = _sc_encode(to_tiles(token_ids), to_tiles(position_ids))
    out = out.transpose(0, 1, 3, 2, 4).reshape(BATCH, NBINS, WORDS)
    return out.transpose(0, 2, 1)
